# baseline (speedup 1.0000x reference)
import functools

import jax
import jax.numpy as jnp
from jax import lax
from jax.experimental import pallas as pl
from jax.experimental.pallas import tpu as pltpu
from jax.experimental.pallas import tpu_sc as plsc

B = 16384
D = 64
CH = 256
CB = 8192
SPLIT_U = 524288
SPLIT_I = 65536


def _pack_body(top_ref, bot_ref, elo_ref, ehi_ref, out_ref):
    cn = (((0,), (0,)), ((), ()))
    t = (lax.dot_general(top_ref[...], elo_ref[...], cn) +
         lax.dot_general(bot_ref[...], ehi_ref[...], cn))
    lo = lax.shift_right_logical(_b16hi(t[:CB // 2]), 16)
    hi = _b16hi(t[CB // 2:])
    out_ref[...] = hi | lo


def _b16hi(x):
    b = lax.bitcast_convert_type(x, jnp.int32)
    r = b + jnp.int32(0x7FFF) + (lax.shift_right_logical(b, 16) & 1)
    return r & jnp.int32(-65536)


def _pack(tT, split, elo, ehi):
    grid = split // CB
    nblk = -(-tT.shape[1] // CB) - 1
    return pl.pallas_call(
        _pack_body,
        grid=(grid,),
        in_specs=[
            pl.BlockSpec((D, CB), lambda j: (0, j)),
            pl.BlockSpec((D, CB), lambda j, g=grid, n=nblk: (0, jnp.minimum(j + g, n))),
            pl.BlockSpec((D, 2 * D), lambda j: (0, 0)),
            pl.BlockSpec((D, 2 * D), lambda j: (0, 0)),
        ],
        out_specs=pl.BlockSpec((CB // 2, 2 * D), lambda j: (j, 0)),
        out_shape=jax.ShapeDtypeStruct((split // 2, 2 * D), jnp.int32),
    )(tT, tT, elo, ehi)


def _sc_gather_body(user_hbm, item_hbm, utP_hbm, itP_hbm, uout_hbm, iout_hbm,
                    idx_u, idx_i, pidx_u, pidx_i, r128_u, r128_i, r64_u,
                    r64_i, sem_u, sem_i, nc, bpw):
    wid = lax.axis_index("s") * nc + lax.axis_index("c")
    base = wid * bpw
    pltpu.sync_copy(user_hbm.at[pl.ds(base, bpw)], idx_u)
    pltpu.sync_copy(item_hbm.at[pl.ds(base, bpw)], idx_i)

    def prep(g, _):
        gb = g * 16
        vu = idx_u[pl.ds(gb, 16)]
        vi = idx_i[pl.ds(gb, 16)]
        ru = vu - jnp.where(vu >= SPLIT_U, SPLIT_U, 0)
        ri = vi - jnp.where(vi >= SPLIT_I, SPLIT_I, 0)
        pidx_u[pl.ds(gb, 16)] = (
            lax.shift_left(lax.shift_right_logical(ru, 13), 12) | (ru & 4095))
        pidx_i[pl.ds(gb, 16)] = (
            lax.shift_left(lax.shift_right_logical(ri, 13), 12) | (ri & 4095))
        return 0

    lax.fori_loop(0, bpw // 16, prep, 0)

    for c in range(bpw // CH):
        cb = c * CH
        cu = pltpu.async_copy(
            utP_hbm.at[pidx_u.at[pl.ds(cb, CH)]], r128_u, sem_u)
        ci = pltpu.async_copy(
            itP_hbm.at[pidx_i.at[pl.ds(cb, CH)]], r128_i, sem_i)
        cu.wait()
        ci.wait()

        def sel(g, _, cb=cb):
            gb = g * 16
            vu = idx_u[pl.ds(cb + gb, 16)]
            vi = idx_i[pl.ds(cb + gb, 16)]
            for jj in range(16):
                row = gb + jj
                hu = jnp.where(vu[jj] >= SPLIT_U, D, 0)
                hi = jnp.where(vi[jj] >= SPLIT_I, D, 0)
                for q in range(4):
                    r64_u[pl.ds(row * 64 + q * 16, 16)] = (
                        r128_u[row, pl.ds(hu + q * 16, 16)])
                    r64_i[pl.ds(row * 64 + q * 16, 16)] = (
                        r128_i[row, pl.ds(hi + q * 16, 16)])
            return 0

        lax.fori_loop(0, CH // 16, sel, 0)
        pltpu.sync_copy(r64_u, uout_hbm.at[pl.ds((base + cb) * 64, CH * 64)])
        pltpu.sync_copy(r64_i, iout_hbm.at[pl.ds((base + cb) * 64, CH * 64)])


@jax.jit
def _sc_gather(user, item, utP, itP):
    info = plsc.get_sparse_core_info()
    nc, ns = info.num_cores, info.num_subcores
    nw = nc * ns
    bpw = B // nw
    mesh = plsc.VectorSubcoreMesh(core_axis_name="c", subcore_axis_name="s")
    body = functools.partial(_sc_gather_body, nc=nc, bpw=bpw)
    k = pl.kernel(
        body,
        out_type=[
            jax.ShapeDtypeStruct((B * D,), jnp.int32),
            jax.ShapeDtypeStruct((B * D,), jnp.int32),
        ],
        mesh=mesh,
        compiler_params=pltpu.CompilerParams(use_tc_tiling_on_sc=True),
        scratch_types=[
            pltpu.VMEM((bpw,), jnp.int32),
            pltpu.VMEM((bpw,), jnp.int32),
            pltpu.VMEM((bpw,), jnp.int32),
            pltpu.VMEM((bpw,), jnp.int32),
            pltpu.VMEM((CH, 128), jnp.int32),
            pltpu.VMEM((CH, 128), jnp.int32),
            pltpu.VMEM((CH * 64,), jnp.int32),
            pltpu.VMEM((CH * 64,), jnp.int32),
            pltpu.SemaphoreType.DMA,
            pltpu.SemaphoreType.DMA,
        ],
    )
    return k(user, item, utP, itP)


def _unpk(v, par):
    lo = lax.bitcast_convert_type(v << 16, jnp.float32)
    hi = lax.bitcast_convert_type(v & jnp.int32(-65536), jnp.float32)
    return lo + (hi - lo) * par


def _mlp_body(u_ref, i_ref, pu_ref, pi_ref, w1a_ref, w1b_ref, b1_ref,
              w2_ref, b2_ref, w3_ref, b3_ref, out_ref):
    u = _unpk(u_ref[...], pu_ref[...])
    i = _unpk(i_ref[...], pi_ref[...])
    h = u @ w1a_ref[...] + i @ w1b_ref[...] + b1_ref[...]
    h = jnp.maximum(h, 0.0)
    h = jnp.maximum(h @ w2_ref[...] + b2_ref[...], 0.0)
    out_ref[...] = h @ w3_ref[...] + b3_ref[...]


@jax.jit
def _mlp(u, i, pu, pi, W1, b1, W2, b2, W3, b3):
    blk = 4096
    grid = B // blk
    w1a = W1[:D]
    w1b = W1[D:]
    full = lambda s: pl.BlockSpec(s, lambda j: (0, 0))
    out = pl.pallas_call(
        _mlp_body,
        grid=(grid,),
        in_specs=[
            pl.BlockSpec((blk, D), lambda j: (j, 0)),
            pl.BlockSpec((blk, D), lambda j: (j, 0)),
            pl.BlockSpec((blk, 1), lambda j: (j, 0)),
            pl.BlockSpec((blk, 1), lambda j: (j, 0)),
            full((D, 64)),
            full((D, 64)),
            full((1, 64)),
            full((64, 32)),
            full((1, 32)),
            full((32, 1)),
            full((1, 1)),
        ],
        out_specs=pl.BlockSpec((blk, 1), lambda j: (j, 0)),
        out_shape=jax.ShapeDtypeStruct((B, 1), jnp.float32),
    )(u, i, pu, pi, w1a, w1b, b1.reshape(1, 64), W2, b2.reshape(1, 32), W3,
      b3.reshape(1, 1))
    return out


def kernel(user, item, user_table, item_table, W1, b1, W2, b2, W3, b3):
    user = user.astype(jnp.int32)
    item = item.astype(jnp.int32)
    elo = jnp.eye(D, 2 * D, dtype=jnp.float32)
    ehi = jnp.eye(D, 2 * D, k=D, dtype=jnp.float32)
    utP = _pack(user_table.T, SPLIT_U, elo, ehi)
    itP = _pack(item_table.T, SPLIT_I, elo, ehi)
    uo, io = _sc_gather(user, item, utP, itP)
    ru = user - jnp.where(user >= SPLIT_U, SPLIT_U, 0)
    ri = item - jnp.where(item >= SPLIT_I, SPLIT_I, 0)
    pu = ((ru >> 12) & 1).astype(jnp.float32).reshape(B, 1)
    pi = ((ri >> 12) & 1).astype(jnp.float32).reshape(B, 1)
    out = _mlp(uo.reshape(B, D), io.reshape(B, D), pu, pi, W1, b1, W2, b2,
               W3, b3)
    return jnp.squeeze(out, axis=-1)


# speedup vs baseline: 1.1523x; 1.1523x over previous
import functools

import jax
import jax.numpy as jnp
from jax import lax
from jax.experimental import pallas as pl
from jax.experimental.pallas import tpu as pltpu
from jax.experimental.pallas import tpu_sc as plsc

B = 16384
D = 64
CH = 256
CB = 8192
SPLIT_U = 524288
SPLIT_I = 65536


def _pack_body(top_ref, bot_ref, elo_ref, ehi_ref, out_ref):
    cn = (((0,), (0,)), ((), ()))
    f32 = jnp.float32
    t = (lax.dot_general(top_ref[...].astype(jnp.bfloat16), elo_ref[...],
                         cn, preferred_element_type=f32) +
         lax.dot_general(bot_ref[...].astype(jnp.bfloat16), ehi_ref[...],
                         cn, preferred_element_type=f32))
    bits = lax.bitcast_convert_type(t, jnp.int32)
    out_ref[...] = bits[CB // 2:] | lax.shift_right_logical(
        bits[:CB // 2], 16)


def _pack(tT, split, elo, ehi):
    grid = split // CB
    nblk = -(-tT.shape[1] // CB) - 1
    return pl.pallas_call(
        _pack_body,
        grid=(grid,),
        compiler_params=pltpu.CompilerParams(
            fuse_transposed_lhs_in_matmul=True),
        in_specs=[
            pl.BlockSpec((D, CB), lambda j: (0, j)),
            pl.BlockSpec((D, CB), lambda j, g=grid, n=nblk: (0, jnp.minimum(j + g, n))),
            pl.BlockSpec((D, 2 * D), lambda j: (0, 0)),
            pl.BlockSpec((D, 2 * D), lambda j: (0, 0)),
        ],
        out_specs=pl.BlockSpec((CB // 2, 2 * D), lambda j: (j, 0)),
        out_shape=jax.ShapeDtypeStruct((split // 2, 2 * D), jnp.int32),
    )(tT, tT, elo, ehi)


def _sc_gather_body(user_hbm, item_hbm, utP_hbm, itP_hbm, uout_hbm, iout_hbm,
                    idx_u, idx_i, pidx_u, pidx_i, r128_u, r128_i, r64_u,
                    r64_i, sem_u, sem_i, nc, bpw):
    wid = lax.axis_index("s") * nc + lax.axis_index("c")
    base = wid * bpw
    pltpu.sync_copy(user_hbm.at[pl.ds(base, bpw)], idx_u)
    pltpu.sync_copy(item_hbm.at[pl.ds(base, bpw)], idx_i)

    def prep(g, _):
        gb = g * 16
        vu = idx_u[pl.ds(gb, 16)]
        vi = idx_i[pl.ds(gb, 16)]
        ru = vu - jnp.where(vu >= SPLIT_U, SPLIT_U, 0)
        ri = vi - jnp.where(vi >= SPLIT_I, SPLIT_I, 0)
        pidx_u[pl.ds(gb, 16)] = (
            lax.shift_left(lax.shift_right_logical(ru, 13), 12) | (ru & 4095))
        pidx_i[pl.ds(gb, 16)] = (
            lax.shift_left(lax.shift_right_logical(ri, 13), 12) | (ri & 4095))
        return 0

    lax.fori_loop(0, bpw // 16, prep, 0)

    for c in range(bpw // CH):
        cb = c * CH
        cu = pltpu.async_copy(
            utP_hbm.at[pidx_u.at[pl.ds(cb, CH)]], r128_u, sem_u)
        ci = pltpu.async_copy(
            itP_hbm.at[pidx_i.at[pl.ds(cb, CH)]], r128_i, sem_i)
        cu.wait()
        ci.wait()

        def sel(g, _, cb=cb):
            gb = g * 16
            vu = idx_u[pl.ds(cb + gb, 16)]
            vi = idx_i[pl.ds(cb + gb, 16)]
            for jj in range(16):
                row = gb + jj
                hu = jnp.where(vu[jj] >= SPLIT_U, D, 0)
                hi = jnp.where(vi[jj] >= SPLIT_I, D, 0)
                for q in range(4):
                    r64_u[pl.ds(row * 64 + q * 16, 16)] = (
                        r128_u[row, pl.ds(hu + q * 16, 16)])
                    r64_i[pl.ds(row * 64 + q * 16, 16)] = (
                        r128_i[row, pl.ds(hi + q * 16, 16)])
            return 0

        lax.fori_loop(0, CH // 16, sel, 0)
        pltpu.sync_copy(r64_u, uout_hbm.at[pl.ds((base + cb) * 64, CH * 64)])
        pltpu.sync_copy(r64_i, iout_hbm.at[pl.ds((base + cb) * 64, CH * 64)])


@jax.jit
def _sc_gather(user, item, utP, itP):
    info = plsc.get_sparse_core_info()
    nc, ns = info.num_cores, info.num_subcores
    nw = nc * ns
    bpw = B // nw
    mesh = plsc.VectorSubcoreMesh(core_axis_name="c", subcore_axis_name="s")
    body = functools.partial(_sc_gather_body, nc=nc, bpw=bpw)
    k = pl.kernel(
        body,
        out_type=[
            jax.ShapeDtypeStruct((B * D,), jnp.int32),
            jax.ShapeDtypeStruct((B * D,), jnp.int32),
        ],
        mesh=mesh,
        compiler_params=pltpu.CompilerParams(use_tc_tiling_on_sc=True),
        scratch_types=[
            pltpu.VMEM((bpw,), jnp.int32),
            pltpu.VMEM((bpw,), jnp.int32),
            pltpu.VMEM((bpw,), jnp.int32),
            pltpu.VMEM((bpw,), jnp.int32),
            pltpu.VMEM((CH, 128), jnp.int32),
            pltpu.VMEM((CH, 128), jnp.int32),
            pltpu.VMEM((CH * 64,), jnp.int32),
            pltpu.VMEM((CH * 64,), jnp.int32),
            pltpu.SemaphoreType.DMA,
            pltpu.SemaphoreType.DMA,
        ],
    )
    return k(user, item, utP, itP)


def _unpk(v, par):
    lo = lax.bitcast_convert_type(v << 16, jnp.float32)
    hi = lax.bitcast_convert_type(v & jnp.int32(-65536), jnp.float32)
    return lo + (hi - lo) * par


def _mlp_body(u_ref, i_ref, pu_ref, pi_ref, w1a_ref, w1b_ref, b1_ref,
              w2_ref, b2_ref, w3_ref, b3_ref, out_ref):
    u = _unpk(u_ref[...], pu_ref[...])
    i = _unpk(i_ref[...], pi_ref[...])
    h = u @ w1a_ref[...] + i @ w1b_ref[...] + b1_ref[...]
    h = jnp.maximum(h, 0.0)
    h = jnp.maximum(h @ w2_ref[...] + b2_ref[...], 0.0)
    out_ref[...] = h @ w3_ref[...] + b3_ref[...]


@jax.jit
def _mlp(u, i, pu, pi, W1, b1, W2, b2, W3, b3):
    blk = 4096
    grid = B // blk
    w1a = W1[:D]
    w1b = W1[D:]
    full = lambda s: pl.BlockSpec(s, lambda j: (0, 0))
    out = pl.pallas_call(
        _mlp_body,
        grid=(grid,),
        in_specs=[
            pl.BlockSpec((blk, D), lambda j: (j, 0)),
            pl.BlockSpec((blk, D), lambda j: (j, 0)),
            pl.BlockSpec((blk, 1), lambda j: (j, 0)),
            pl.BlockSpec((blk, 1), lambda j: (j, 0)),
            full((D, 64)),
            full((D, 64)),
            full((1, 64)),
            full((64, 32)),
            full((1, 32)),
            full((32, 1)),
            full((1, 1)),
        ],
        out_specs=pl.BlockSpec((blk, 1), lambda j: (j, 0)),
        out_shape=jax.ShapeDtypeStruct((B, 1), jnp.float32),
    )(u, i, pu, pi, w1a, w1b, b1.reshape(1, 64), W2, b2.reshape(1, 32), W3,
      b3.reshape(1, 1))
    return out


def kernel(user, item, user_table, item_table, W1, b1, W2, b2, W3, b3):
    user = user.astype(jnp.int32)
    item = item.astype(jnp.int32)
    elo = jnp.eye(D, 2 * D, dtype=jnp.bfloat16)
    ehi = jnp.eye(D, 2 * D, k=D, dtype=jnp.bfloat16)
    utP = _pack(user_table.T, SPLIT_U, elo, ehi)
    itP = _pack(item_table.T, SPLIT_I, elo, ehi)
    uo, io = _sc_gather(user, item, utP, itP)
    ru = user - jnp.where(user >= SPLIT_U, SPLIT_U, 0)
    ri = item - jnp.where(item >= SPLIT_I, SPLIT_I, 0)
    pu = ((ru >> 12) & 1).astype(jnp.float32).reshape(B, 1)
    pi = ((ri >> 12) & 1).astype(jnp.float32).reshape(B, 1)
    out = _mlp(uo.reshape(B, D), io.reshape(B, D), pu, pi, W1, b1, W2, b2,
               W3, b3)
    return jnp.squeeze(out, axis=-1)


# speedup vs baseline: 1.1913x; 1.0339x over previous
import functools

import jax
import jax.numpy as jnp
from jax import lax
from jax.experimental import pallas as pl
from jax.experimental.pallas import tpu as pltpu
from jax.experimental.pallas import tpu_sc as plsc

B = 16384
D = 64
CH = 256
CB = 8192
SPLIT_U = 524288
SPLIT_I = 65536


def _pack_body(top_ref, bot_ref, elo_ref, ehi_ref, out_ref):
    cn = (((0,), (0,)), ((), ()))
    f32 = jnp.float32
    t = (lax.dot_general(top_ref[...].astype(jnp.bfloat16), elo_ref[...],
                         cn, preferred_element_type=f32) +
         lax.dot_general(bot_ref[...].astype(jnp.bfloat16), ehi_ref[...],
                         cn, preferred_element_type=f32))
    bits = lax.bitcast_convert_type(t, jnp.int32)
    out_ref[...] = bits[CB // 2:] | lax.shift_right_logical(
        bits[:CB // 2], 16)


def _pack(tT, split, elo, ehi):
    grid = split // CB
    nblk = -(-tT.shape[1] // CB) - 1
    return pl.pallas_call(
        _pack_body,
        grid=(grid,),
        compiler_params=pltpu.CompilerParams(
            fuse_transposed_lhs_in_matmul=True),
        in_specs=[
            pl.BlockSpec((D, CB), lambda j: (0, j)),
            pl.BlockSpec((D, CB), lambda j, g=grid, n=nblk: (0, jnp.minimum(j + g, n))),
            pl.BlockSpec((D, 2 * D), lambda j: (0, 0)),
            pl.BlockSpec((D, 2 * D), lambda j: (0, 0)),
        ],
        out_specs=pl.BlockSpec((CB // 2, 2 * D), lambda j: (j, 0)),
        out_shape=jax.ShapeDtypeStruct((split // 2, 2 * D), jnp.int32),
    )(tT, tT, elo, ehi)


def _sc_gather_body(ids_hbm, tP_hbm, out_hbm, idx, pidx, r128, r64, sem,
                    nc, bpw, split):
    wid = lax.axis_index("s") * nc + lax.axis_index("c")
    base = wid * bpw
    pltpu.sync_copy(ids_hbm.at[pl.ds(base, bpw)], idx)

    def prep(g, _):
        gb = g * 16
        v = idx[pl.ds(gb, 16)]
        r = v - jnp.where(v >= split, split, 0)
        pidx[pl.ds(gb, 16)] = (
            lax.shift_left(lax.shift_right_logical(r, 13), 12) | (r & 4095))
        return 0

    lax.fori_loop(0, bpw // 16, prep, 0)

    for c in range(bpw // CH):
        cb = c * CH
        pltpu.async_copy(
            tP_hbm.at[pidx.at[pl.ds(cb, CH)]], r128, sem).wait()

        def sel(g, _, cb=cb):
            gb = g * 16
            v = idx[pl.ds(cb + gb, 16)]
            for jj in range(16):
                row = gb + jj
                h = jnp.where(v[jj] >= split, D, 0)
                for q in range(4):
                    r64[pl.ds(row * 64 + q * 16, 16)] = (
                        r128[row, pl.ds(h + q * 16, 16)])
            return 0

        lax.fori_loop(0, CH // 16, sel, 0)
        pltpu.sync_copy(r64, out_hbm.at[pl.ds((base + cb) * 64, CH * 64)])


@jax.jit
def _sc_gather(ids, tP, split):
    info = plsc.get_sparse_core_info()
    nc, ns = info.num_cores, info.num_subcores
    nw = nc * ns
    bpw = B // nw
    mesh = plsc.VectorSubcoreMesh(core_axis_name="c", subcore_axis_name="s")
    body = functools.partial(_sc_gather_body, nc=nc, bpw=bpw, split=split)
    k = pl.kernel(
        body,
        out_type=jax.ShapeDtypeStruct((B * D,), jnp.int32),
        mesh=mesh,
        compiler_params=pltpu.CompilerParams(use_tc_tiling_on_sc=True),
        scratch_types=[
            pltpu.VMEM((bpw,), jnp.int32),
            pltpu.VMEM((bpw,), jnp.int32),
            pltpu.VMEM((CH, 128), jnp.int32),
            pltpu.VMEM((CH * 64,), jnp.int32),
            pltpu.SemaphoreType.DMA,
        ],
    )
    return k(ids, tP)


def _dotb(x, w):
    return lax.dot_general(x.astype(jnp.bfloat16), w,
                           (((1,), (0,)), ((), ())),
                           preferred_element_type=jnp.float32)


def _unpk(v, par):
    lo = lax.bitcast_convert_type(v << 16, jnp.float32)
    hi = lax.bitcast_convert_type(v & jnp.int32(-65536), jnp.float32)
    return lo + (hi - lo) * par


def _mlp_body(u_ref, i_ref, pu_ref, pi_ref, w1a_ref, w1b_ref, b1_ref,
              w2_ref, b2_ref, w3_ref, b3_ref, out_ref):
    u = _unpk(u_ref[...], pu_ref[...])
    i = _unpk(i_ref[...], pi_ref[...])
    h = _dotb(u, w1a_ref[...]) + _dotb(i, w1b_ref[...]) + b1_ref[...]
    h = jnp.maximum(h, 0.0)
    h = jnp.maximum(_dotb(h, w2_ref[...]) + b2_ref[...], 0.0)
    out_ref[...] = _dotb(h, w3_ref[...]) + b3_ref[...]


@jax.jit
def _mlp(u, i, pu, pi, W1, b1, W2, b2, W3, b3):
    blk = 4096
    grid = B // blk
    w1a = W1[:D].astype(jnp.bfloat16)
    w1b = W1[D:].astype(jnp.bfloat16)
    W2 = W2.astype(jnp.bfloat16)
    W3 = W3.astype(jnp.bfloat16)
    full = lambda s: pl.BlockSpec(s, lambda j: (0, 0))
    out = pl.pallas_call(
        _mlp_body,
        grid=(grid,),
        in_specs=[
            pl.BlockSpec((blk, D), lambda j: (j, 0)),
            pl.BlockSpec((blk, D), lambda j: (j, 0)),
            pl.BlockSpec((blk, 1), lambda j: (j, 0)),
            pl.BlockSpec((blk, 1), lambda j: (j, 0)),
            full((D, 64)),
            full((D, 64)),
            full((1, 64)),
            full((64, 32)),
            full((1, 32)),
            full((32, 1)),
            full((1, 1)),
        ],
        out_specs=pl.BlockSpec((blk, 1), lambda j: (j, 0)),
        out_shape=jax.ShapeDtypeStruct((B, 1), jnp.float32),
    )(u, i, pu, pi, w1a, w1b, b1.reshape(1, 64), W2, b2.reshape(1, 32), W3,
      b3.reshape(1, 1))
    return out


def kernel(user, item, user_table, item_table, W1, b1, W2, b2, W3, b3):
    user = user.astype(jnp.int32)
    item = item.astype(jnp.int32)
    elo = jnp.eye(D, 2 * D, dtype=jnp.bfloat16)
    ehi = jnp.eye(D, 2 * D, k=D, dtype=jnp.bfloat16)
    utP = _pack(user_table.T, SPLIT_U, elo, ehi)
    itP = _pack(item_table.T, SPLIT_I, elo, ehi)
    uo = _sc_gather(user, utP, SPLIT_U)
    io = _sc_gather(item, itP, SPLIT_I)
    ru = user - jnp.where(user >= SPLIT_U, SPLIT_U, 0)
    ri = item - jnp.where(item >= SPLIT_I, SPLIT_I, 0)
    pu = ((ru >> 12) & 1).astype(jnp.float32).reshape(B, 1)
    pi = ((ri >> 12) & 1).astype(jnp.float32).reshape(B, 1)
    out = _mlp(uo.reshape(B, D), io.reshape(B, D), pu, pi, W1, b1, W2, b2,
               W3, b3)
    return jnp.squeeze(out, axis=-1)


# speedup vs baseline: 1.1920x; 1.0005x over previous
import functools

import jax
import jax.numpy as jnp
from jax import lax
from jax.experimental import pallas as pl
from jax.experimental.pallas import tpu as pltpu
from jax.experimental.pallas import tpu_sc as plsc

B = 16384
D = 64
CH = 256
CB = 8192
SPLIT_U = 524288
SPLIT_I = 65536


def _pack_body(top_ref, bot_ref, elo_ref, ehi_ref, out_ref):
    cn = (((0,), (0,)), ((), ()))
    f32 = jnp.float32
    t = (lax.dot_general(top_ref[...].astype(jnp.bfloat16), elo_ref[...],
                         cn, preferred_element_type=f32) +
         lax.dot_general(bot_ref[...].astype(jnp.bfloat16), ehi_ref[...],
                         cn, preferred_element_type=f32))
    bits = lax.bitcast_convert_type(t, jnp.int32)
    out_ref[...] = bits[CB // 2:] | lax.shift_right_logical(
        bits[:CB // 2], 16)


def _pack(tT, split, elo, ehi):
    grid = split // CB
    nblk = -(-tT.shape[1] // CB) - 1
    return pl.pallas_call(
        _pack_body,
        grid=(grid,),
        compiler_params=pltpu.CompilerParams(
            fuse_transposed_lhs_in_matmul=True),
        in_specs=[
            pl.BlockSpec((D, CB), lambda j: (0, j)),
            pl.BlockSpec((D, CB), lambda j, g=grid, n=nblk: (0, jnp.minimum(j + g, n))),
            pl.BlockSpec((D, 2 * D), lambda j: (0, 0)),
            pl.BlockSpec((D, 2 * D), lambda j: (0, 0)),
        ],
        out_specs=pl.BlockSpec((CB // 2, 2 * D), lambda j: (j, 0)),
        out_shape=jax.ShapeDtypeStruct((split // 2, 2 * D), jnp.int32),
    )(tT, tT, elo, ehi)


def _sc_gather_body(ids_hbm, tP_hbm, out_hbm, idx, pidx, r128a, r128b, r64,
                    sem_a, sem_b, nc, bpw, split):
    r128s = (r128a, r128b)
    sems = (sem_a, sem_b)
    wid = lax.axis_index("s") * nc + lax.axis_index("c")
    base = wid * bpw
    pltpu.sync_copy(ids_hbm.at[pl.ds(base, bpw)], idx)

    def prep(g, _):
        gb = g * 16
        v = idx[pl.ds(gb, 16)]
        r = v - jnp.where(v >= split, split, 0)
        pidx[pl.ds(gb, 16)] = (
            lax.shift_left(lax.shift_right_logical(r, 13), 12) | (r & 4095))
        return 0

    lax.fori_loop(0, bpw // 16, prep, 0)

    copies = [
        pltpu.async_copy(
            tP_hbm.at[pidx.at[pl.ds(c * CH, CH)]], r128s[c], sems[c])
        for c in range(bpw // CH)
    ]
    for c in range(bpw // CH):
        cb = c * CH
        copies[c].wait()
        r128 = r128s[c]

        def sel(g, _, cb=cb, r128=r128):
            gb = g * 16
            v = idx[pl.ds(cb + gb, 16)]
            for jj in range(16):
                row = gb + jj
                h = jnp.where(v[jj] >= split, D, 0)
                for q in range(4):
                    r64[pl.ds(row * 64 + q * 16, 16)] = (
                        r128[row, pl.ds(h + q * 16, 16)])
            return 0

        lax.fori_loop(0, CH // 16, sel, 0)
        pltpu.sync_copy(r64, out_hbm.at[pl.ds((base + cb) * 64, CH * 64)])


@jax.jit
def _sc_gather(ids, tP, split):
    info = plsc.get_sparse_core_info()
    nc, ns = info.num_cores, info.num_subcores
    nw = nc * ns
    bpw = B // nw
    mesh = plsc.VectorSubcoreMesh(core_axis_name="c", subcore_axis_name="s")
    body = functools.partial(_sc_gather_body, nc=nc, bpw=bpw, split=split)
    k = pl.kernel(
        body,
        out_type=jax.ShapeDtypeStruct((B * D,), jnp.int32),
        mesh=mesh,
        compiler_params=pltpu.CompilerParams(use_tc_tiling_on_sc=True),
        scratch_types=[
            pltpu.VMEM((bpw,), jnp.int32),
            pltpu.VMEM((bpw,), jnp.int32),
            pltpu.VMEM((CH, 128), jnp.int32),
            pltpu.VMEM((CH, 128), jnp.int32),
            pltpu.VMEM((CH * 64,), jnp.int32),
            pltpu.SemaphoreType.DMA,
            pltpu.SemaphoreType.DMA,
        ],
    )
    return k(ids, tP)


def _dotb(x, w):
    return lax.dot_general(x.astype(jnp.bfloat16), w,
                           (((1,), (0,)), ((), ())),
                           preferred_element_type=jnp.float32)


def _unpk(v, par):
    lo = lax.bitcast_convert_type(v << 16, jnp.float32)
    hi = lax.bitcast_convert_type(v & jnp.int32(-65536), jnp.float32)
    return lo + (hi - lo) * par


def _mlp_body(u_ref, i_ref, pu_ref, pi_ref, w1a_ref, w1b_ref, b1_ref,
              w2_ref, b2_ref, w3_ref, b3_ref, out_ref):
    u = _unpk(u_ref[...], pu_ref[...])
    i = _unpk(i_ref[...], pi_ref[...])
    h = _dotb(u, w1a_ref[...]) + _dotb(i, w1b_ref[...]) + b1_ref[...]
    h = jnp.maximum(h, 0.0)
    h = jnp.maximum(_dotb(h, w2_ref[...]) + b2_ref[...], 0.0)
    out_ref[...] = _dotb(h, w3_ref[...]) + b3_ref[...]


@jax.jit
def _mlp(u, i, pu, pi, W1, b1, W2, b2, W3, b3):
    blk = 4096
    grid = B // blk
    w1a = W1[:D].astype(jnp.bfloat16)
    w1b = W1[D:].astype(jnp.bfloat16)
    W2 = W2.astype(jnp.bfloat16)
    W3 = W3.astype(jnp.bfloat16)
    full = lambda s: pl.BlockSpec(s, lambda j: (0, 0))
    out = pl.pallas_call(
        _mlp_body,
        grid=(grid,),
        in_specs=[
            pl.BlockSpec((blk, D), lambda j: (j, 0)),
            pl.BlockSpec((blk, D), lambda j: (j, 0)),
            pl.BlockSpec((blk, 1), lambda j: (j, 0)),
            pl.BlockSpec((blk, 1), lambda j: (j, 0)),
            full((D, 64)),
            full((D, 64)),
            full((1, 64)),
            full((64, 32)),
            full((1, 32)),
            full((32, 1)),
            full((1, 1)),
        ],
        out_specs=pl.BlockSpec((blk, 1), lambda j: (j, 0)),
        out_shape=jax.ShapeDtypeStruct((B, 1), jnp.float32),
    )(u, i, pu, pi, w1a, w1b, b1.reshape(1, 64), W2, b2.reshape(1, 32), W3,
      b3.reshape(1, 1))
    return out


def kernel(user, item, user_table, item_table, W1, b1, W2, b2, W3, b3):
    user = user.astype(jnp.int32)
    item = item.astype(jnp.int32)
    elo = jnp.eye(D, 2 * D, dtype=jnp.bfloat16)
    ehi = jnp.eye(D, 2 * D, k=D, dtype=jnp.bfloat16)
    utP = _pack(user_table.T, SPLIT_U, elo, ehi)
    itP = _pack(item_table.T, SPLIT_I, elo, ehi)
    uo = _sc_gather(user, utP, SPLIT_U)
    io = _sc_gather(item, itP, SPLIT_I)
    ru = user - jnp.where(user >= SPLIT_U, SPLIT_U, 0)
    ri = item - jnp.where(item >= SPLIT_I, SPLIT_I, 0)
    pu = ((ru >> 12) & 1).astype(jnp.float32).reshape(B, 1)
    pi = ((ri >> 12) & 1).astype(jnp.float32).reshape(B, 1)
    out = _mlp(uo.reshape(B, D), io.reshape(B, D), pu, pi, W1, b1, W2, b2,
               W3, b3)
    return jnp.squeeze(out, axis=-1)


# speedup vs baseline: 1.2392x; 1.0397x over previous
import functools

import jax
import jax.numpy as jnp
from jax import lax
from jax.experimental import pallas as pl
from jax.experimental.pallas import tpu as pltpu
from jax.experimental.pallas import tpu_sc as plsc

B = 16384
D = 64
CH = 256
CB = 16384
SPLIT_U = 524288
SPLIT_I = 65536


def _pack_body(top_ref, bot_ref, elo_ref, ehi_ref, out_ref):
    cn = (((0,), (0,)), ((), ()))
    f32 = jnp.float32
    t = (lax.dot_general(top_ref[...].astype(jnp.bfloat16), elo_ref[...],
                         cn, preferred_element_type=f32) +
         lax.dot_general(bot_ref[...].astype(jnp.bfloat16), ehi_ref[...],
                         cn, preferred_element_type=f32))
    bits = lax.bitcast_convert_type(t, jnp.int32)
    out_ref[...] = bits[CB // 2:] | lax.shift_right_logical(
        bits[:CB // 2], 16)


def _pack(tT, split, elo, ehi):
    grid = split // CB
    nblk = -(-tT.shape[1] // CB) - 1
    return pl.pallas_call(
        _pack_body,
        grid=(grid,),
        compiler_params=pltpu.CompilerParams(
            fuse_transposed_lhs_in_matmul=True),
        in_specs=[
            pl.BlockSpec((D, CB), lambda j: (0, j)),
            pl.BlockSpec((D, CB), lambda j, g=grid, n=nblk: (0, jnp.minimum(j + g, n))),
            pl.BlockSpec((D, 2 * D), lambda j: (0, 0)),
            pl.BlockSpec((D, 2 * D), lambda j: (0, 0)),
        ],
        out_specs=pl.BlockSpec((CB // 2, 2 * D), lambda j: (j, 0)),
        out_shape=jax.ShapeDtypeStruct((split // 2, 2 * D), jnp.int32),
    )(tT, tT, elo, ehi)


def _sc_gather_body(ids_hbm, tP_hbm, out_hbm, idx, pidx, r128a, r128b, r64,
                    sem_a, sem_b, nc, bpw, split):
    r128s = (r128a, r128b)
    sems = (sem_a, sem_b)
    wid = lax.axis_index("s") * nc + lax.axis_index("c")
    base = wid * bpw
    pltpu.sync_copy(ids_hbm.at[pl.ds(base, bpw)], idx)

    def prep(g, _):
        gb = g * 16
        v = idx[pl.ds(gb, 16)]
        r = v - jnp.where(v >= split, split, 0)
        pidx[pl.ds(gb, 16)] = (
            lax.shift_left(lax.shift_right_logical(r, 14), 13) | (r & 8191))
        return 0

    lax.fori_loop(0, bpw // 16, prep, 0)

    copies = [
        pltpu.async_copy(
            tP_hbm.at[pidx.at[pl.ds(c * CH, CH)]], r128s[c], sems[c])
        for c in range(bpw // CH)
    ]
    for c in range(bpw // CH):
        cb = c * CH
        copies[c].wait()
        r128 = r128s[c]

        def sel(g, _, cb=cb, r128=r128):
            gb = g * 16
            v = idx[pl.ds(cb + gb, 16)]
            for jj in range(16):
                row = gb + jj
                h = jnp.where(v[jj] >= split, D, 0)
                for q in range(4):
                    r64[pl.ds(row * 64 + q * 16, 16)] = (
                        r128[row, pl.ds(h + q * 16, 16)])
            return 0

        lax.fori_loop(0, CH // 16, sel, 0)
        pltpu.sync_copy(r64, out_hbm.at[pl.ds((base + cb) * 64, CH * 64)])


@jax.jit
def _sc_gather(ids, tP, split):
    info = plsc.get_sparse_core_info()
    nc, ns = info.num_cores, info.num_subcores
    nw = nc * ns
    bpw = B // nw
    mesh = plsc.VectorSubcoreMesh(core_axis_name="c", subcore_axis_name="s")
    body = functools.partial(_sc_gather_body, nc=nc, bpw=bpw, split=split)
    k = pl.kernel(
        body,
        out_type=jax.ShapeDtypeStruct((B * D,), jnp.int32),
        mesh=mesh,
        compiler_params=pltpu.CompilerParams(use_tc_tiling_on_sc=True),
        scratch_types=[
            pltpu.VMEM((bpw,), jnp.int32),
            pltpu.VMEM((bpw,), jnp.int32),
            pltpu.VMEM((CH, 128), jnp.int32),
            pltpu.VMEM((CH, 128), jnp.int32),
            pltpu.VMEM((CH * 64,), jnp.int32),
            pltpu.SemaphoreType.DMA,
            pltpu.SemaphoreType.DMA,
        ],
    )
    return k(ids, tP)


def _dotb(x, w):
    return lax.dot_general(x.astype(jnp.bfloat16), w,
                           (((1,), (0,)), ((), ())),
                           preferred_element_type=jnp.float32)


def _unpk(v, par):
    lo = lax.bitcast_convert_type(v << 16, jnp.float32)
    hi = lax.bitcast_convert_type(v & jnp.int32(-65536), jnp.float32)
    return lo + (hi - lo) * par


def _mlp_body(u_ref, i_ref, pu_ref, pi_ref, w1a_ref, w1b_ref, b1_ref,
              w2_ref, b2_ref, w3_ref, b3_ref, out_ref):
    u = _unpk(u_ref[...], pu_ref[...])
    i = _unpk(i_ref[...], pi_ref[...])
    h = _dotb(u, w1a_ref[...]) + _dotb(i, w1b_ref[...]) + b1_ref[...]
    h = jnp.maximum(h, 0.0)
    h = jnp.maximum(_dotb(h, w2_ref[...]) + b2_ref[...], 0.0)
    out_ref[...] = _dotb(h, w3_ref[...]) + b3_ref[...]


@jax.jit
def _mlp(u, i, pu, pi, W1, b1, W2, b2, W3, b3):
    blk = 4096
    grid = B // blk
    w1a = W1[:D].astype(jnp.bfloat16)
    w1b = W1[D:].astype(jnp.bfloat16)
    W2 = W2.astype(jnp.bfloat16)
    W3 = W3.astype(jnp.bfloat16)
    full = lambda s: pl.BlockSpec(s, lambda j: (0, 0))
    out = pl.pallas_call(
        _mlp_body,
        grid=(grid,),
        in_specs=[
            pl.BlockSpec((blk, D), lambda j: (j, 0)),
            pl.BlockSpec((blk, D), lambda j: (j, 0)),
            pl.BlockSpec((blk, 1), lambda j: (j, 0)),
            pl.BlockSpec((blk, 1), lambda j: (j, 0)),
            full((D, 64)),
            full((D, 64)),
            full((1, 64)),
            full((64, 32)),
            full((1, 32)),
            full((32, 1)),
            full((1, 1)),
        ],
        out_specs=pl.BlockSpec((blk, 1), lambda j: (j, 0)),
        out_shape=jax.ShapeDtypeStruct((B, 1), jnp.float32),
    )(u, i, pu, pi, w1a, w1b, b1.reshape(1, 64), W2, b2.reshape(1, 32), W3,
      b3.reshape(1, 1))
    return out


def kernel(user, item, user_table, item_table, W1, b1, W2, b2, W3, b3):
    user = user.astype(jnp.int32)
    item = item.astype(jnp.int32)
    elo = jnp.eye(D, 2 * D, dtype=jnp.bfloat16)
    ehi = jnp.eye(D, 2 * D, k=D, dtype=jnp.bfloat16)
    utP = _pack(user_table.T, SPLIT_U, elo, ehi)
    itP = _pack(item_table.T, SPLIT_I, elo, ehi)
    uo = _sc_gather(user, utP, SPLIT_U)
    io = _sc_gather(item, itP, SPLIT_I)
    ru = user - jnp.where(user >= SPLIT_U, SPLIT_U, 0)
    ri = item - jnp.where(item >= SPLIT_I, SPLIT_I, 0)
    pu = ((ru >> 13) & 1).astype(jnp.float32).reshape(B, 1)
    pi = ((ri >> 13) & 1).astype(jnp.float32).reshape(B, 1)
    out = _mlp(uo.reshape(B, D), io.reshape(B, D), pu, pi, W1, b1, W2, b2,
               W3, b3)
    return jnp.squeeze(out, axis=-1)
